# baseline (speedup 1.0000x reference)
import functools

import jax
import jax.numpy as jnp
from jax import lax
from jax.experimental import pallas as pl
from jax.experimental.pallas import tpu as pltpu
from jax.experimental.pallas import tpu_sc as plsc

_V = 100000
_D = 64
_C = 20
_H = 128

_BV = 4096
_NB = pl.cdiv(_V, _BV)
_BV2 = 8192
_NB2 = pl.cdiv(_V, _BV2)


@functools.cache
def _make_sc_gather():
    @functools.partial(
        pl.kernel,
        mesh=plsc.VectorSubcoreMesh(core_axis_name="c", subcore_axis_name="s"),
        out_type=jax.ShapeDtypeStruct((_C, _D), jnp.float32),
        scratch_types=[
            pltpu.VMEM((_C,), jnp.int32),
            pltpu.VMEM((_C, _D), jnp.float32),
            pltpu.SemaphoreType.DMA,
        ],
        compiler_params=pltpu.CompilerParams(use_tc_tiling_on_sc=False),
    )
    def _sc_gather(idx_hbm, table_hbm, out_hbm, idx_v, rows_v, sem):
        wid = lax.axis_index("s") * 2 + lax.axis_index("c")

        @pl.when(wid == 0)
        def _():
            pltpu.sync_copy(idx_hbm, idx_v)
            pltpu.async_copy(table_hbm.at[idx_v], rows_v, sem).wait()
            pltpu.sync_copy(rows_v, out_hbm)

    return _sc_gather


def _mlp_body(e_ref, w1_ref, b1_ref, w2_ref, b2_ref, o_ref, logz_ref,
              h_s, m_s, s_s):
    i = pl.program_id(0)

    @pl.when(i == 0)
    def _():
        h = jnp.dot(e_ref[...], w1_ref[...],
                    preferred_element_type=jnp.float32) + b1_ref[...]
        h_s[...] = jnp.maximum(h, 0.0)
        m_s[0] = -jnp.inf
        s_s[0] = 0.0

    o_blk = jnp.dot(h_s[...], w2_ref[...],
                    preferred_element_type=jnp.float32) + b2_ref[...]
    cols = i * _BV + lax.broadcasted_iota(jnp.int32, (1, _BV), 1)
    o_blk = jnp.where(cols < _V, o_blk, -jnp.inf)
    o_ref[...] = o_blk

    m_old = m_s[0]
    m_new = jnp.maximum(m_old, jnp.max(o_blk))
    s_s[0] = s_s[0] * jnp.exp(m_old - m_new) + jnp.sum(jnp.exp(o_blk - m_new))
    m_s[0] = m_new

    @pl.when(i == _NB - 1)
    def _():
        logz_ref[0] = m_s[0] + jnp.log(s_s[0])


def _norm_body(o_ref, logz_ref, out_ref):
    out_ref[...] = o_ref[...] - logz_ref[0]


def kernel(inputs, emb, W1, b1, W2, b2):
    idx = inputs.astype(jnp.int32)
    e = _make_sc_gather()(idx, emb)
    e2 = e.reshape(1, _C * _D)

    o, logz = pl.pallas_call(
        _mlp_body,
        grid=(_NB,),
        in_specs=[
            pl.BlockSpec((1, _C * _D), lambda i: (0, 0)),
            pl.BlockSpec((_C * _D, _H), lambda i: (0, 0)),
            pl.BlockSpec((1, _H), lambda i: (0, 0)),
            pl.BlockSpec((_H, _BV), lambda i: (0, i)),
            pl.BlockSpec((1, _BV), lambda i: (0, i)),
        ],
        out_specs=[
            pl.BlockSpec((1, _BV), lambda i: (0, i)),
            pl.BlockSpec(memory_space=pltpu.SMEM),
        ],
        out_shape=[
            jax.ShapeDtypeStruct((1, _V), jnp.float32),
            jax.ShapeDtypeStruct((1,), jnp.float32),
        ],
        scratch_shapes=[
            pltpu.VMEM((1, _H), jnp.float32),
            pltpu.SMEM((1,), jnp.float32),
            pltpu.SMEM((1,), jnp.float32),
        ],
        compiler_params=pltpu.CompilerParams(
            dimension_semantics=("arbitrary",),
        ),
    )(e2, W1, b1.reshape(1, _H), W2, b2.reshape(1, _V))

    log_probs = pl.pallas_call(
        _norm_body,
        grid=(_NB2,),
        in_specs=[
            pl.BlockSpec((1, _BV2), lambda i: (0, i)),
            pl.BlockSpec(memory_space=pltpu.SMEM),
        ],
        out_specs=pl.BlockSpec((1, _BV2), lambda i: (0, i)),
        out_shape=jax.ShapeDtypeStruct((1, _V), jnp.float32),
    )(o, logz)

    return log_probs


# speedup vs baseline: 1.3132x; 1.3132x over previous
import functools

import jax
import jax.numpy as jnp
from jax import lax
from jax.experimental import pallas as pl
from jax.experimental.pallas import tpu as pltpu
from jax.experimental.pallas import tpu_sc as plsc

_V = 100000
_D = 64
_C = 20
_H = 128

_BV = 4096
_NB = pl.cdiv(_V, _BV)
_BV2 = 8192
_NB2 = pl.cdiv(_V, _BV2)


def _sc_gather(jdx, table3):
    jdx_scalars = [jdx[c] for c in range(_C)]

    @functools.partial(
        pl.kernel,
        mesh=plsc.ScalarSubcoreMesh(axis_name="c", num_cores=2),
        out_type=jax.ShapeDtypeStruct((_C, 8, _D), jnp.float32),
        scratch_types=[
            pltpu.SemaphoreType.DMA,
        ],
    )
    def _body(table3_hbm, out_hbm, sem):
        @pl.when(lax.axis_index("c") == 0)
        def _():
            copies = []
            for c in range(_C):
                copies.append(
                    pltpu.async_copy(
                        table3_hbm.at[jdx_scalars[c]], out_hbm.at[c], sem))
            for cp in copies:
                cp.wait()

    return _body(table3)


def _mlp_body(tiles_ref, oh_ref, w1_ref, b1_ref, w2_ref, b2_ref,
              o_ref, logz_ref, h_s, m_s, s_s):
    i = pl.program_id(0)

    @pl.when(i == 0)
    def _():
        e_sel = jnp.sum(tiles_ref[...] * oh_ref[...][:, :, None], axis=1)
        h = b1_ref[...]
        for c in range(_C):
            h = h + jnp.dot(e_sel[c:c + 1, :], w1_ref[c * _D:(c + 1) * _D, :],
                            preferred_element_type=jnp.float32)
        h_s[...] = jnp.maximum(h, 0.0)
        m_s[0] = -jnp.inf
        s_s[0] = 0.0

    o_blk = jnp.dot(h_s[...], w2_ref[...],
                    preferred_element_type=jnp.float32) + b2_ref[...]
    cols = i * _BV + lax.broadcasted_iota(jnp.int32, (1, _BV), 1)
    o_blk = jnp.where(cols < _V, o_blk, -jnp.inf)
    o_ref[...] = o_blk

    m_old = m_s[0]
    m_new = jnp.maximum(m_old, jnp.max(o_blk))
    s_s[0] = s_s[0] * jnp.exp(m_old - m_new) + jnp.sum(jnp.exp(o_blk - m_new))
    m_s[0] = m_new

    @pl.when(i == _NB - 1)
    def _():
        logz_ref[0] = m_s[0] + jnp.log(s_s[0])


def _norm_body(o_ref, logz_ref, out_ref):
    out_ref[...] = o_ref[...] - logz_ref[0]


def kernel(inputs, emb, W1, b1, W2, b2):
    idx = inputs.astype(jnp.int32)
    jdx = idx // 8
    onehot = (idx[:, None] % 8 ==
              jnp.arange(8, dtype=jnp.int32)[None, :]).astype(jnp.float32)

    tiles = _sc_gather(jdx, emb.reshape(_V // 8, 8, _D))

    o, logz = pl.pallas_call(
        _mlp_body,
        grid=(_NB,),
        in_specs=[
            pl.BlockSpec((_C, 8, _D), lambda i: (0, 0, 0)),
            pl.BlockSpec((_C, 8), lambda i: (0, 0)),
            pl.BlockSpec((_C * _D, _H), lambda i: (0, 0)),
            pl.BlockSpec((1, _H), lambda i: (0, 0)),
            pl.BlockSpec((_H, _BV), lambda i: (0, i)),
            pl.BlockSpec((1, _BV), lambda i: (0, i)),
        ],
        out_specs=[
            pl.BlockSpec((1, _BV), lambda i: (0, i)),
            pl.BlockSpec(memory_space=pltpu.SMEM),
        ],
        out_shape=[
            jax.ShapeDtypeStruct((1, _V), jnp.float32),
            jax.ShapeDtypeStruct((1,), jnp.float32),
        ],
        scratch_shapes=[
            pltpu.VMEM((1, _H), jnp.float32),
            pltpu.SMEM((1,), jnp.float32),
            pltpu.SMEM((1,), jnp.float32),
        ],
        compiler_params=pltpu.CompilerParams(
            dimension_semantics=("arbitrary",),
        ),
    )(tiles, onehot, W1, b1.reshape(1, _H), W2, b2.reshape(1, _V))

    log_probs = pl.pallas_call(
        _norm_body,
        grid=(_NB2,),
        in_specs=[
            pl.BlockSpec((1, _BV2), lambda i: (0, i)),
            pl.BlockSpec(memory_space=pltpu.SMEM),
        ],
        out_specs=pl.BlockSpec((1, _BV2), lambda i: (0, i)),
        out_shape=jax.ShapeDtypeStruct((1, _V), jnp.float32),
    )(o, logz)

    return log_probs
